# Initial kernel scaffold; baseline (speedup 1.0000x reference)
import jax
import jax.numpy as jnp
from jax.experimental import pallas as pl


def kernel(x, table):
    raise NotImplementedError("write your pallas kernel here")


# speedup vs baseline: 1.0166x; 1.0166x over previous
import jax
import jax.numpy as jnp
from jax import lax
from jax.experimental import pallas as pl
from jax.experimental.pallas import tpu as pltpu
from jax.experimental.pallas import tpu_sc as plsc

_VOCAB = 1000
_B = 1024
_L = 50
_NUM_IDX = _B * _L
_NC, _NS = 2, 16
_NW = _NC * _NS
_PER_W = _NUM_IDX // _NW
_W = 64
_CHUNKS = _PER_W // _W


def kernel(x, table):
    idx = x.reshape(_NW, _CHUNKS, _W).astype(jnp.int32)

    mesh = plsc.VectorSubcoreMesh(core_axis_name="c", subcore_axis_name="s")

    @pl.kernel(
        out_type=jax.ShapeDtypeStruct((_NUM_IDX, _VOCAB), table.dtype),
        mesh=mesh,
        compiler_params=pltpu.CompilerParams(use_tc_tiling_on_sc=False),
        scratch_types=[
            pltpu.VMEM((_CHUNKS, _W), jnp.int32),
            pltpu.VMEM((_W, _VOCAB), jnp.float32),
            pltpu.SemaphoreType.DMA,
        ],
    )
    def gather_kernel(table_hbm, idx_hbm, out_hbm, idx_v, rows_v, sem):
        wid = lax.axis_index("s") * _NC + lax.axis_index("c")
        base = wid * _PER_W
        pltpu.sync_copy(idx_hbm.at[wid], idx_v)

        @pl.loop(0, _CHUNKS)
        def _(c):
            pltpu.async_copy(table_hbm.at[idx_v.at[c]], rows_v, sem).wait()
            pltpu.sync_copy(rows_v, out_hbm.at[pl.ds(base + c * _W, _W)])

    out = gather_kernel(table, idx)
    return out.reshape(_B, _L, _VOCAB)


# speedup vs baseline: 1.0276x; 1.0108x over previous
import jax
import jax.numpy as jnp
from jax import lax
from jax.experimental import pallas as pl
from jax.experimental.pallas import tpu as pltpu
from jax.experimental.pallas import tpu_sc as plsc

_VOCAB = 1000
_B = 1024
_L = 50
_NUM_IDX = _B * _L
_NC, _NS = 2, 16
_NW = _NC * _NS
_PER_W = _NUM_IDX // _NW
_W = 40
_NBUF = 2
_CHUNKS = _PER_W // _W


def kernel(x, table):
    idx = x.reshape(_NW, _CHUNKS, _W).astype(jnp.int32)

    mesh = plsc.VectorSubcoreMesh(core_axis_name="c", subcore_axis_name="s")

    @pl.kernel(
        out_type=jax.ShapeDtypeStruct((_NUM_IDX, _VOCAB), table.dtype),
        mesh=mesh,
        compiler_params=pltpu.CompilerParams(use_tc_tiling_on_sc=False),
        scratch_types=[
            pltpu.VMEM((_CHUNKS, _W), jnp.int32),
            pltpu.VMEM((_W, _VOCAB), jnp.float32),
            pltpu.VMEM((_W, _VOCAB), jnp.float32),
            pltpu.SemaphoreType.DMA,
            pltpu.SemaphoreType.DMA,
            pltpu.SemaphoreType.DMA,
            pltpu.SemaphoreType.DMA,
        ],
    )
    def gather_kernel(table_hbm, idx_hbm, out_hbm, idx_v, buf0, buf1,
                      gs0, gs1, ws0, ws1):
        bufs = (buf0, buf1)
        gsems = (gs0, gs1)
        wsems = (ws0, ws1)
        wid = lax.axis_index("s") * _NC + lax.axis_index("c")
        base = wid * _PER_W
        pltpu.sync_copy(idx_hbm.at[wid], idx_v)

        def gather_desc(c, b):
            return pltpu.make_async_copy(
                table_hbm.at[idx_v.at[c]], bufs[b], gsems[b])

        def write_desc(c, b):
            return pltpu.make_async_copy(
                bufs[b], out_hbm.at[pl.ds(base + c * _W, _W)], wsems[b])

        gather_desc(0, 0).start()

        @pl.loop(0, _CHUNKS, step=_NBUF)
        def _(g):
            for b in range(_NBUF):
                c = g + b
                gather_desc(c, b).wait()
                write_desc(c, b).start()
                nb = (b + 1) % _NBUF

                @pl.when(c + 1 < _CHUNKS)
                def _():
                    @pl.when(c + 1 - _NBUF >= 0)
                    def _():
                        write_desc(c + 1 - _NBUF, nb).wait()

                    gather_desc(c + 1, nb).start()

        for k in range(_NBUF):
            c = _CHUNKS - _NBUF + k
            write_desc(c, c % _NBUF).wait()

    out = gather_kernel(table, idx)
    return out.reshape(_B, _L, _VOCAB)


# speedup vs baseline: 1.0302x; 1.0026x over previous
import jax
import jax.numpy as jnp
from jax import lax
from jax.experimental import pallas as pl
from jax.experimental.pallas import tpu as pltpu
from jax.experimental.pallas import tpu_sc as plsc

_VOCAB = 1000
_B = 1024
_L = 50
_NC, _NS = 2, 16
_NW = _NC * _NS
_BPW = _B // _NW
_NBUF = 2


def kernel(x, table):
    idx = x.astype(jnp.int32)

    mesh = plsc.VectorSubcoreMesh(core_axis_name="c", subcore_axis_name="s")

    @pl.kernel(
        out_type=jax.ShapeDtypeStruct((_B, _L, _VOCAB), table.dtype),
        mesh=mesh,
        compiler_params=pltpu.CompilerParams(use_tc_tiling_on_sc=False),
        scratch_types=[
            pltpu.VMEM((_BPW, _L), jnp.int32),
            pltpu.VMEM((_L, _VOCAB), jnp.float32),
            pltpu.VMEM((_L, _VOCAB), jnp.float32),
            pltpu.SemaphoreType.DMA,
            pltpu.SemaphoreType.DMA,
            pltpu.SemaphoreType.DMA,
            pltpu.SemaphoreType.DMA,
        ],
    )
    def gather_kernel(table_hbm, idx_hbm, out_hbm, idx_v, buf0, buf1,
                      gs0, gs1, ws0, ws1):
        bufs = (buf0, buf1)
        gsems = (gs0, gs1)
        wsems = (ws0, ws1)
        wid = lax.axis_index("s") * _NC + lax.axis_index("c")
        base = wid * _BPW
        pltpu.sync_copy(idx_hbm.at[pl.ds(base, _BPW)], idx_v)

        def gather_desc(c, b):
            return pltpu.make_async_copy(
                table_hbm.at[idx_v.at[c]], bufs[b], gsems[b])

        def write_desc(c, b):
            return pltpu.make_async_copy(
                bufs[b], out_hbm.at[base + c], wsems[b])

        gather_desc(0, 0).start()

        @pl.loop(0, _BPW, step=_NBUF)
        def _(g):
            for b in range(_NBUF):
                c = g + b
                gather_desc(c, b).wait()
                write_desc(c, b).start()
                nb = (b + 1) % _NBUF

                @pl.when(c + 1 < _BPW)
                def _():
                    @pl.when(c + 1 - _NBUF >= 0)
                    def _():
                        write_desc(c + 1 - _NBUF, nb).wait()

                    gather_desc(c + 1, nb).start()

        for k in range(_NBUF):
            c = _BPW - _NBUF + k
            write_desc(c, c % _NBUF).wait()

    return gather_kernel(table, idx)


# speedup vs baseline: 1.2349x; 1.1987x over previous
import jax
import jax.numpy as jnp
from jax import lax
from jax.experimental import pallas as pl
from jax.experimental.pallas import tpu as pltpu
from jax.experimental.pallas import tpu_sc as plsc

_VOCAB = 1000
_B = 1024
_L = 50
_VP = 1024
_NC, _NS = 2, 16
_NW = _NC * _NS
_NSEG = _B // 16
_SPLIT = 29


def kernel(x, table):
    xt = jnp.transpose(x).reshape(_L * _B).astype(jnp.int32)
    tblT = jnp.pad(jnp.transpose(table), ((0, _VP - _VOCAB), (0, _VP - _VOCAB)))

    mesh = plsc.VectorSubcoreMesh(core_axis_name="c", subcore_axis_name="s")

    @pl.kernel(
        out_type=jax.ShapeDtypeStruct((_L, _VOCAB, _B), table.dtype),
        mesh=mesh,
        compiler_params=pltpu.CompilerParams(
            use_tc_tiling_on_sc=True, needs_layout_passes=False),
        scratch_types=[
            pltpu.VMEM((32, _VP), jnp.float32),
            pltpu.VMEM((_L * _B,), jnp.int32),
            pltpu.VMEM((8, _B), jnp.float32),
            pltpu.VMEM((8, _B), jnp.float32),
            pltpu.SemaphoreType.DMA,
            pltpu.SemaphoreType.DMA,
        ],
    )
    def lookup_kernel(tbl_hbm, xt_hbm, q_hbm, tbl_v, idx_v, ob0, ob1, ws0, ws1):
        obufs = (ob0, ob1)
        wsems = (ws0, ws1)
        wid = lax.axis_index("s") * _NC + lax.axis_index("c")
        small = wid >= _SPLIT
        nvt = jnp.where(small, 3, 4)
        vt0 = jnp.where(small, 4 * _SPLIT + 3 * (wid - _SPLIT), 4 * wid)
        v0 = vt0 * 8

        pltpu.sync_copy(tbl_hbm.at[pl.ds(v0, 32)], tbl_v)
        pltpu.sync_copy(xt_hbm, idx_v)

        total = _L * nvt

        def compute(l, j, ob):
            rows = [jnp.full((16,), j * 8 + vi, jnp.int32) for vi in range(8)]

            @pl.loop(0, _NSEG)
            def _(s):
                idx16 = idx_v[pl.ds(l * _B + s * 16, 16)]
                for vi in range(8):
                    vals = plsc.load_gather(tbl_v, [rows[vi], idx16])
                    ob[vi, pl.ds(s * 16, 16)] = vals

        def write_desc(l, j, p):
            return pltpu.make_async_copy(
                obufs[p], q_hbm.at[l].at[pl.ds(v0 + j * 8, 8)], wsems[p])

        @pl.loop(0, total, step=2)
        def _(u):
            for p in range(2):
                uu = u + p
                l = lax.div(uu, nvt)
                j = uu - l * nvt

                @pl.when(uu >= 2)
                def _():
                    write_desc(l, j, p).wait()

                compute(l, j, obufs[p])
                write_desc(l, j, p).start()

        for p in range(2):
            lastu = total - 2 + p
            ll = lax.div(lastu, nvt)
            write_desc(ll, lastu - ll * nvt, p).wait()

    q = lookup_kernel(tblT, xt)
    return jnp.transpose(q, (2, 0, 1))


# speedup vs baseline: 1.2830x; 1.0390x over previous
import jax
import jax.numpy as jnp
from jax import lax
from jax.experimental import pallas as pl
from jax.experimental.pallas import tpu as pltpu
from jax.experimental.pallas import tpu_sc as plsc

_VOCAB = 1000
_B = 1024
_L = 50
_VP = 1024
_NC, _NS = 2, 16
_NW = _NC * _NS
_NSEG = _B // 16
_SPLIT = 29


def kernel(x, table):
    xt = jnp.transpose(x).reshape(_L * _B).astype(jnp.int32)
    tblT = jnp.pad(
        jnp.transpose(table), ((0, _VP - _VOCAB), (0, _VP - _VOCAB))
    ).reshape(_VP * _VP)

    mesh = plsc.VectorSubcoreMesh(core_axis_name="c", subcore_axis_name="s")

    @pl.kernel(
        out_type=jax.ShapeDtypeStruct((_L, _VOCAB, _B), table.dtype),
        mesh=mesh,
        compiler_params=pltpu.CompilerParams(
            use_tc_tiling_on_sc=True, needs_layout_passes=False),
        scratch_types=[
            pltpu.VMEM((32 * _VP,), jnp.float32),
            pltpu.VMEM((_L * _B,), jnp.int32),
            pltpu.VMEM((8, _B), jnp.float32),
            pltpu.VMEM((8, _B), jnp.float32),
            pltpu.SemaphoreType.DMA,
            pltpu.SemaphoreType.DMA,
        ],
    )
    def lookup_kernel(tbl_hbm, xt_hbm, q_hbm, tbl_v, idx_v, ob0, ob1, ws0, ws1):
        obufs = (ob0, ob1)
        wsems = (ws0, ws1)
        wid = lax.axis_index("s") * _NC + lax.axis_index("c")
        small = wid >= _SPLIT
        nvt = jnp.where(small, 3, 4)
        vt0 = jnp.where(small, 4 * _SPLIT + 3 * (wid - _SPLIT), 4 * wid)
        v0 = vt0 * 8

        pltpu.sync_copy(tbl_hbm.at[pl.ds(v0 * _VP, 32 * _VP)], tbl_v)
        pltpu.sync_copy(xt_hbm, idx_v)

        total = _L * nvt

        def compute(l, j, ob):
            rowoffs = [jnp.full((16,), (j * 8 + vi) * _VP, jnp.int32)
                       for vi in range(8)]

            @pl.loop(0, _NSEG)
            def _(s):
                idx16 = idx_v[pl.ds(l * _B + s * 16, 16)]
                for vi in range(8):
                    vals = plsc.load_gather(tbl_v, [idx16 + rowoffs[vi]])
                    ob[vi, pl.ds(s * 16, 16)] = vals

        def write_desc(l, j, p):
            return pltpu.make_async_copy(
                obufs[p], q_hbm.at[l].at[pl.ds(v0 + j * 8, 8)], wsems[p])

        @pl.loop(0, total, step=2)
        def _(u):
            for p in range(2):
                uu = u + p
                l = lax.div(uu, nvt)
                j = uu - l * nvt

                @pl.when(uu >= 2)
                def _():
                    write_desc(l, j, p).wait()

                compute(l, j, obufs[p])
                write_desc(l, j, p).start()

        for p in range(2):
            lastu = total - 2 + p
            ll = lax.div(lastu, nvt)
            write_desc(ll, lastu - ll * nvt, p).wait()

    q = lookup_kernel(tblT, xt)
    return jnp.transpose(q, (2, 0, 1))


# speedup vs baseline: 1.3301x; 1.0367x over previous
import jax
import jax.numpy as jnp
from jax import lax
from jax.experimental import pallas as pl
from jax.experimental.pallas import tpu as pltpu
from jax.experimental.pallas import tpu_sc as plsc

_VOCAB = 1000
_B = 1024
_L = 50
_VP = 1024
_NC, _NS = 2, 16
_NW = _NC * _NS
_NSEG = _B // 16
_SPLIT = 29


def kernel(x, table):
    xt = jnp.transpose(x).reshape(_L * _B).astype(jnp.int32)
    tblT = jnp.pad(
        jnp.transpose(table), ((0, _VP - _VOCAB), (0, _VP - _VOCAB))
    ).reshape(_VP * _VP)

    mesh = plsc.VectorSubcoreMesh(core_axis_name="c", subcore_axis_name="s")

    @pl.kernel(
        out_type=jax.ShapeDtypeStruct((_L, _VOCAB, _B), table.dtype),
        mesh=mesh,
        compiler_params=pltpu.CompilerParams(
            use_tc_tiling_on_sc=True, needs_layout_passes=False),
        scratch_types=[
            pltpu.VMEM((32 * _VP,), jnp.float32),
            pltpu.VMEM((_L * _B,), jnp.int32),
            pltpu.VMEM((8, _B), jnp.float32),
            pltpu.VMEM((8, _B), jnp.float32),
            pltpu.SemaphoreType.DMA,
            pltpu.SemaphoreType.DMA,
        ],
    )
    def lookup_kernel(tbl_hbm, xt_hbm, q_hbm, tbl_v, idx_v, ob0, ob1, ws0, ws1):
        obufs = (ob0, ob1)
        wsems = (ws0, ws1)
        wid = lax.axis_index("s") * _NC + lax.axis_index("c")
        small = wid >= _SPLIT
        nvt = jnp.where(small, 3, 4)
        vt0 = jnp.where(small, 4 * _SPLIT + 3 * (wid - _SPLIT), 4 * wid)
        v0 = vt0 * 8

        pltpu.sync_copy(tbl_hbm.at[pl.ds(v0 * _VP, 32 * _VP)], tbl_v)
        pltpu.sync_copy(xt_hbm, idx_v)

        total = _L * nvt

        def compute(l, j, ob):
            rows = [tbl_v.at[pl.ds((j * 8 + vi) * _VP, _VP)] for vi in range(8)]

            @pl.loop(0, _NSEG, step=2)
            def _(s):
                for t in range(2):
                    idx16 = idx_v[pl.ds(l * _B + (s + t) * 16, 16)]
                    for vi in range(8):
                        vals = plsc.load_gather(rows[vi], [idx16])
                        ob[vi, pl.ds((s + t) * 16, 16)] = vals

        def write_desc(l, j, p):
            return pltpu.make_async_copy(
                obufs[p], q_hbm.at[l].at[pl.ds(v0 + j * 8, 8)], wsems[p])

        @pl.loop(0, total, step=2)
        def _(u):
            for p in range(2):
                uu = u + p
                l = lax.div(uu, nvt)
                j = uu - l * nvt

                @pl.when(uu >= 2)
                def _():
                    write_desc(l, j, p).wait()

                compute(l, j, obufs[p])
                write_desc(l, j, p).start()

        for p in range(2):
            lastu = total - 2 + p
            ll = lax.div(lastu, nvt)
            write_desc(ll, lastu - ll * nvt, p).wait()

    q = lookup_kernel(tblT, xt)
    return jnp.transpose(q, (2, 0, 1))
